# baseline (speedup 1.0000x reference)
import functools

import jax
import jax.numpy as jnp
from jax import lax
from jax.experimental import pallas as pl
from jax.experimental.pallas import tpu as pltpu
from jax.experimental.pallas import tpu_sc as plsc

_B, _C = 64, 4
_NF, _MS = 1024, 4096
_J = 1024
_NB, _L = 2, 8192

_P = _B * _C
_NC, _NS, _LN = 2, 16, 16
_NW = _NC * _NS
_PPW = _P // _NW
_SL = _NB * _L
_S2 = 2 * _SL
_CL = 2048
_CL2 = 2 * _CL
_NCHUNK = _SL // _CL
_J2 = 2 * _J
_NF2 = 2 * _NF


def _sc_body(ifp, tfp, isp, tsp, xia, ksa, ifs, tfs, iss, tss,
             harm_out, scat_out,
             tifb, ttfb, fgb, sbi, sbt, xib, ksb, accH, accS):
    wid = lax.axis_index("s") * _NC + lax.axis_index("c")
    lane = lax.iota(jnp.int32, _LN)
    zero = jnp.zeros((_LN,), jnp.float32)

    def scat_plane(k, acc):
        p = wid * _PPW + k
        pltpu.sync_copy(ifs.at[p], sbi.at[pl.ds(0, _NF2)])
        pltpu.sync_copy(tfs.at[p], sbt.at[pl.ds(0, _NF2)])

        def f_iter(t, a):
            ie = 2 * (t * _LN + lane)
            t0 = plsc.load_gather(sbt, [ie])
            t1 = plsc.load_gather(sbt, [ie + 1])
            i0 = plsc.load_gather(sbi, [ie])
            i1 = plsc.load_gather(sbi, [ie + 1])
            v = t1 - i1 - 2.0 * t0 * (t0 - i0)
            return a + v * v

        acc = lax.fori_loop(0, _NF // _LN, f_iter, acc)

        pltpu.sync_copy(iss.at[p], sbi)
        pltpu.sync_copy(tss.at[p], sbt)

        def s_iter(t, a):
            s = pl.ds(t * _LN, _LN)
            d = sbt[s] - sbi[s]
            return a + d * d

        return lax.fori_loop(0, _MS // _LN, s_iter, acc)

    accS[...] = lax.fori_loop(0, _PPW, scat_plane, zero)

    pltpu.sync_copy(xia, xib)
    pltpu.sync_copy(ksa, ksb)

    def plane_loop(k, acc):
        p = wid * _PPW + k
        pltpu.sync_copy(ifp.at[p], tifb)
        pltpu.sync_copy(tfp.at[p], ttfb)

        def tbl_iter(t, c2):
            s = pl.ds(t * _LN, _LN)
            fgb[s] = tifb[s] - ttfb[s]
            return c2

        lax.fori_loop(0, _J2 // _LN, tbl_iter, 0)

        def chunk_loop(cn, a2):
            off = cn * _CL2
            pltpu.sync_copy(isp.at[p, pl.ds(off, _CL2)], sbi)
            pltpu.sync_copy(tsp.at[p, pl.ds(off, _CL2)], sbt)
            lbase = cn * _CL

            def hot(t, a3):
                lg = lbase + t * _LN + lane
                e2 = 2 * lg
                v0 = plsc.load_gather(xib, [e2])
                v1 = plsc.load_gather(xib, [e2 + 1])
                k0 = plsc.load_gather(ksb, [e2])
                k1 = plsc.load_gather(ksb, [e2 + 1])
                m = jnp.where((k0 == 0) & (k1 == 0), 1.0, 0.0)
                i0 = 2 * v0
                i1 = 2 * v1
                f0r = plsc.load_gather(fgb, [i0])
                f0i = plsc.load_gather(fgb, [i0 + 1])
                f1r = plsc.load_gather(fgb, [i1])
                f1i = plsc.load_gather(fgb, [i1 + 1])
                g0r = plsc.load_gather(ttfb, [i0])
                g0i = plsc.load_gather(ttfb, [i0 + 1])
                g1r = plsc.load_gather(ttfb, [i1])
                g1i = plsc.load_gather(ttfb, [i1 + 1])
                er = f0r * g1r + f0i * g1i - g0r * f1r - g0i * f1i
                ei = f0i * g1r - f0r * g1i + g0r * f1i - g0i * f1r
                ls2 = 2 * (t * _LN + lane)
                sr = plsc.load_gather(sbi, [ls2]) - plsc.load_gather(sbt, [ls2])
                si = plsc.load_gather(sbi, [ls2 + 1]) - plsc.load_gather(sbt, [ls2 + 1])
                dr = sr - m * er
                di = si - m * ei
                return a3 + dr * dr + di * di

            return lax.fori_loop(0, _CL // _LN, hot, a2)

        return lax.fori_loop(0, _NCHUNK, chunk_loop, acc)

    accH[...] = lax.fori_loop(0, _PPW, plane_loop, zero)

    pltpu.sync_copy(accH, harm_out.at[wid])
    pltpu.sync_copy(accS, scat_out.at[wid])


_sc_call = functools.partial(
    pl.kernel,
    compiler_params=pltpu.CompilerParams(needs_layout_passes=False),
    out_type=(
        jax.ShapeDtypeStruct((_NW, _LN), jnp.float32),
        jax.ShapeDtypeStruct((_NW, _LN), jnp.float32),
    ),
    mesh=plsc.VectorSubcoreMesh(
        core_axis_name="c", subcore_axis_name="s",
        num_cores=_NC, num_subcores=_NS,
    ),
    scratch_types=[
        pltpu.VMEM((_J2,), jnp.float32),
        pltpu.VMEM((_J2,), jnp.float32),
        pltpu.VMEM((_J2,), jnp.float32),
        pltpu.VMEM((_CL2,), jnp.float32),
        pltpu.VMEM((_CL2,), jnp.float32),
        pltpu.VMEM((2 * _SL,), jnp.int32),
        pltpu.VMEM((2 * _SL,), jnp.int32),
        pltpu.VMEM((_LN,), jnp.float32),
        pltpu.VMEM((_LN,), jnp.float32),
    ],
)(_sc_body)


def kernel(i_f_scat, i_f_phe, i_s_scat, i_s_phe,
           t_f_scat, t_f_phe, t_s_scat, t_s_phe, xi_idx, ks):
    ifp = i_f_phe.reshape(_P, _J2)
    tfp = t_f_phe.reshape(_P, _J2)
    isp = i_s_phe.reshape(_P, _S2)
    tsp = t_s_phe.reshape(_P, _S2)
    xia = xi_idx.reshape(2 * _SL).astype(jnp.int32)
    ksa = ks.reshape(2 * _SL).astype(jnp.int32)
    ifs = i_f_scat.reshape(_P, _NF2)
    tfs = t_f_scat.reshape(_P, _NF2)
    iss = i_s_scat.reshape(_P, _MS)
    tss = t_s_scat.reshape(_P, _MS)

    harm, scat = _sc_call(ifp, tfp, isp, tsp, xia, ksa, ifs, tfs, iss, tss)
    loss_scat = jnp.sum(scat) / (_B * _C)
    loss_harm = jnp.sum(harm) / (_B * _C * _SL)
    return (loss_scat, loss_harm)


# speedup vs baseline: 32.9125x; 32.9125x over previous
import functools

import jax
import jax.numpy as jnp
from jax import lax
from jax.experimental import pallas as pl
from jax.experimental.pallas import tpu as pltpu
from jax.experimental.pallas import tpu_sc as plsc

_B, _C = 64, 4
_NF, _MS = 1024, 4096
_J = 1024
_NB, _L = 2, 8192

_P = _B * _C
_NC, _NS, _LN = 2, 16, 16
_NW = _NC * _NS
_PPW = _P // _NW
_SL = _NB * _L
_S2 = 2 * _SL
_CL = 2048
_CL2 = 2 * _CL
_NCHUNK = _SL // _CL
_J2 = 2 * _J
_NF2 = 2 * _NF
_GPC = _CL // _LN


def _sc_body(ifp, tfp, isp, tsp, xia, ksa, ifs, tfs, iss, tss,
             harm_out, scat_out,
             tifb, ttfb, fgb, sbi0, sbt0, sbi1, sbt1, xib, ksb,
             accH, accS, semi0, semt0, semi1, semt1):
    wid = lax.axis_index("s") * _NC + lax.axis_index("c")
    lane = lax.iota(jnp.int32, _LN)
    zero = jnp.zeros((_LN,), jnp.float32)

    def scat_plane(k, acc):
        p = wid * _PPW + k
        pltpu.sync_copy(ifs.at[pl.ds(p * _NF2, _NF2)], sbi0.at[pl.ds(0, _NF2)])
        pltpu.sync_copy(tfs.at[pl.ds(p * _NF2, _NF2)], sbt0.at[pl.ds(0, _NF2)])

        @plsc.parallel_loop(0, _NF // _LN, unroll=4, carry=acc)
        def f_iter(t, a):
            offr = (t // 8) * 256 + (t % 8) * _LN
            t0 = sbt0[pl.ds(offr, _LN)]
            t1 = sbt0[pl.ds(offr + 128, _LN)]
            i0 = sbi0[pl.ds(offr, _LN)]
            i1 = sbi0[pl.ds(offr + 128, _LN)]
            v = t1 - i1 - 2.0 * t0 * (t0 - i0)
            return a + v * v

        return f_iter

    accS0 = lax.fori_loop(0, _PPW, scat_plane, zero)

    def scat_b(h, acc):
        b = wid * 2 + h

        def blk(q, a):
            off = b * (4 * _MS) + q * _CL2
            pltpu.sync_copy(iss.at[pl.ds(off, _CL2)], sbi0)
            pltpu.sync_copy(tss.at[pl.ds(off, _CL2)], sbt0)

            @plsc.parallel_loop(0, _CL2 // _LN, unroll=8, carry=a)
            def s_iter(t, a2):
                s = pl.ds(t * _LN, _LN)
                d = sbt0[s] - sbi0[s]
                return a2 + d * d

            return s_iter

        return lax.fori_loop(0, 4, blk, acc)

    accS[...] = lax.fori_loop(0, 2, scat_b, accS0)
    pltpu.sync_copy(accS, scat_out.at[wid])

    pltpu.sync_copy(xia, xib)
    pltpu.sync_copy(ksa, ksb)

    bufs = ((sbi0, sbt0, semi0, semt0), (sbi1, sbt1, semi1, semt1))

    def start_chunk(p, cn, buf):
        bi, bt, si, st = buf
        pltpu.async_copy(isp.at[pl.ds(p * _S2 + cn * _CL2, _CL2)], bi, si)
        pltpu.async_copy(tsp.at[pl.ds(p * _S2 + cn * _CL2, _CL2)], bt, st)

    def wait_chunk(p, cn, buf):
        bi, bt, si, st = buf
        pltpu.make_async_copy(isp.at[pl.ds(p * _S2 + cn * _CL2, _CL2)], bi, si).wait()
        pltpu.make_async_copy(tsp.at[pl.ds(p * _S2 + cn * _CL2, _CL2)], bt, st).wait()

    def plane_loop(k, acc):
        p = wid * _PPW + k
        start_chunk(p, 0, bufs[0])
        pltpu.sync_copy(ifp.at[pl.ds(p * _J2, _J2)], tifb)
        pltpu.sync_copy(tfp.at[pl.ds(p * _J2, _J2)], ttfb)

        @plsc.parallel_loop(0, _J2 // _LN, unroll=8)
        def tbl_iter(t):
            s = pl.ds(t * _LN, _LN)
            fgb[s] = tifb[s] - ttfb[s]

        for cn in range(_NCHUNK):
            bi, bt = bufs[cn % 2][0], bufs[cn % 2][1]
            wait_chunk(p, cn, bufs[cn % 2])
            if cn + 1 < _NCHUNK:
                start_chunk(p, cn + 1, bufs[(cn + 1) % 2])

            def hot(t, a, bi=bi, bt=bt, cn=cn):
                g = cn * _GPC + t
                offx = (g // 512) * (2 * _SL // _NB) + ((g % 512) // 8) * 256 \
                    + (g % 8) * _LN
                v0 = xib[pl.ds(offx, _LN)]
                v1 = xib[pl.ds(offx + 128, _LN)]
                k0 = ksb[pl.ds(offx, _LN)]
                k1 = ksb[pl.ds(offx + 128, _LN)]
                m = jnp.where((k0 == 0) & (k1 == 0), 1.0, 0.0)
                i0 = v0 + ((v0 >> 7) << 7)
                i1 = v1 + ((v1 >> 7) << 7)
                f0r = plsc.load_gather(fgb, [i0])
                f0i = plsc.load_gather(fgb, [i0 + 128])
                f1r = plsc.load_gather(fgb, [i1])
                f1i = plsc.load_gather(fgb, [i1 + 128])
                g0r = plsc.load_gather(ttfb, [i0])
                g0i = plsc.load_gather(ttfb, [i0 + 128])
                g1r = plsc.load_gather(ttfb, [i1])
                g1i = plsc.load_gather(ttfb, [i1 + 128])
                er = f0r * g1r + f0i * g1i - g0r * f1r - g0i * f1i
                ei = f0i * g1r - f0r * g1i + g0r * f1i - g0i * f1r
                offs = (t // 8) * 256 + (t % 8) * _LN
                sr = bi[pl.ds(offs, _LN)] - bt[pl.ds(offs, _LN)]
                si = bi[pl.ds(offs + 128, _LN)] - bt[pl.ds(offs + 128, _LN)]
                dr = sr - m * er
                di = si - m * ei
                return a + dr * dr + di * di

            acc = plsc.parallel_loop(0, _GPC, unroll=4, carry=acc)(hot)
        return acc

    accH[...] = lax.fori_loop(0, _PPW, plane_loop, zero)
    pltpu.sync_copy(accH, harm_out.at[wid])


_sc_call = functools.partial(
    pl.kernel,
    compiler_params=pltpu.CompilerParams(needs_layout_passes=False),
    out_type=(
        jax.ShapeDtypeStruct((_NW, _LN), jnp.float32),
        jax.ShapeDtypeStruct((_NW, _LN), jnp.float32),
    ),
    mesh=plsc.VectorSubcoreMesh(
        core_axis_name="c", subcore_axis_name="s",
        num_cores=_NC, num_subcores=_NS,
    ),
    scratch_types=[
        pltpu.VMEM((_J2,), jnp.float32),
        pltpu.VMEM((_J2,), jnp.float32),
        pltpu.VMEM((_J2,), jnp.float32),
        pltpu.VMEM((_CL2,), jnp.float32),
        pltpu.VMEM((_CL2,), jnp.float32),
        pltpu.VMEM((_CL2,), jnp.float32),
        pltpu.VMEM((_CL2,), jnp.float32),
        pltpu.VMEM((2 * _SL,), jnp.int32),
        pltpu.VMEM((2 * _SL,), jnp.int32),
        pltpu.VMEM((_LN,), jnp.float32),
        pltpu.VMEM((_LN,), jnp.float32),
        pltpu.SemaphoreType.DMA,
        pltpu.SemaphoreType.DMA,
        pltpu.SemaphoreType.DMA,
        pltpu.SemaphoreType.DMA,
    ],
)(_sc_body)


def _flat_cplx(x):
    n = x.shape[2]
    return x.reshape(_B, _C, n // 128, 128, 2).transpose(0, 1, 2, 4, 3).reshape(-1)


def kernel(i_f_scat, i_f_phe, i_s_scat, i_s_phe,
           t_f_scat, t_f_phe, t_s_scat, t_s_phe, xi_idx, ks):
    ifp = _flat_cplx(i_f_phe)
    tfp = _flat_cplx(t_f_phe)
    isp = _flat_cplx(i_s_phe)
    tsp = _flat_cplx(t_s_phe)
    ifs = _flat_cplx(i_f_scat)
    tfs = _flat_cplx(t_f_scat)
    iss = i_s_scat.reshape(_B, _C, _MS // 128, 128).transpose(0, 2, 1, 3).reshape(-1)
    tss = t_s_scat.reshape(_B, _C, _MS // 128, 128).transpose(0, 2, 1, 3).reshape(-1)
    xia = xi_idx.astype(jnp.int32).reshape(_NB, _L // 128, 128, 2) \
        .transpose(0, 1, 3, 2).reshape(-1)
    ksa = ks.astype(jnp.int32).reshape(_NB, _L // 128, 128, 2) \
        .transpose(0, 1, 3, 2).reshape(-1)

    harm, scat = _sc_call(ifp, tfp, isp, tsp, xia, ksa, ifs, tfs, iss, tss)
    loss_scat = jnp.sum(scat) / (_B * _C)
    loss_harm = jnp.sum(harm) / (_B * _C * _SL)
    return (loss_scat, loss_harm)
